# Initial kernel scaffold; baseline (speedup 1.0000x reference)
import jax
import jax.numpy as jnp
from jax.experimental import pallas as pl


def kernel(in_feat, adj, W1, b1, W2, b2):
    raise NotImplementedError("write your pallas kernel here")


# speedup vs baseline: 18.8227x; 18.8227x over previous
import functools

import jax
import jax.numpy as jnp
from jax import lax
from jax.experimental import pallas as pl
from jax.experimental.pallas import tpu as pltpu
from jax.experimental.pallas import tpu_sc as plsc

N = 10000
E = 320000
D_IN = 128
D_HID = 512
NCLS = 4
DP = 16

NC = 2
NS = 16
NW = NC * NS
C = 128
EP = 327680
EW = EP // NW
NCH = EW // C

NP = 10240
RPS = NP // NS
ZR = 128
NZ = RPS // ZR


def _fill_const(ref, nrows, d, val):
  vec = jnp.full((16,), val, jnp.float32)

  def body(r, carry):
    for cc in range(d // 16):
      ref[r, pl.ds(cc * 16, 16)] = vec
    return carry

  lax.fori_loop(0, nrows, body, 0)


def _make_sc_agg(d, gather):
  mesh = plsc.VectorSubcoreMesh(core_axis_name="c", subcore_axis_name="s")
  out_type = jax.ShapeDtypeStruct((NC, NP, d), jnp.float32)
  scratch_types = [
      pltpu.VMEM((NCH, C), jnp.int32),
      pltpu.VMEM((C, d), jnp.float32),
      pltpu.VMEM_SHARED((NP, d), jnp.float32),
      pltpu.SemaphoreType.DMA,
  ]
  if gather:
    scratch_types.insert(0, pltpu.VMEM((NCH, C), jnp.int32))

  def body(*refs):
    if gather:
      (tab_hbm, src_hbm, dst_hbm, out_hbm,
       srci, dsti, rows, acc, sem) = refs
    else:
      (dst_hbm, out_hbm, dsti, rows, acc, sem) = refs
    c = lax.axis_index("c")
    s = lax.axis_index("s")
    wid = c * NS + s

    pltpu.sync_copy(dst_hbm.at[wid], dsti)
    if gather:
      pltpu.sync_copy(src_hbm.at[wid], srci)

    _fill_const(rows, ZR, d, 0.0)

    def zcopy(k, carry):
      pltpu.sync_copy(rows, acc.at[pl.ds(s * RPS + k * ZR, ZR)])
      return carry

    lax.fori_loop(0, NZ, zcopy, 0)
    plsc.subcore_barrier()
    if not gather:
      _fill_const(rows, C, d, 1.0)

    def step(j, carry):
      if gather:
        pltpu.async_copy(tab_hbm.at[srci.at[j]], rows, sem).wait()
      pltpu.sync_copy(rows, acc.at[dsti.at[j]], add=True)
      return carry

    lax.fori_loop(0, NCH, step, 0)
    plsc.subcore_barrier()

    def ocopy(k, carry):
      r0 = s * RPS + k * ZR
      pltpu.sync_copy(acc.at[pl.ds(r0, ZR)], rows)
      pltpu.sync_copy(rows, out_hbm.at[c, pl.ds(r0, ZR)])
      return carry

    lax.fori_loop(0, NZ, ocopy, 0)

  params = pltpu.CompilerParams(use_tc_tiling_on_sc=(d == 128))
  return functools.partial(
      pl.kernel, mesh=mesh, out_type=out_type,
      scratch_types=scratch_types, compiler_params=params)(body)


_sc_deg = _make_sc_agg(DP, gather=False)
_sc_agg128 = _make_sc_agg(D_IN, gather=True)
_sc_agg16 = _make_sc_agg(DP, gather=True)

BT = 400
GT = N // BT


def _prep_body(d0, d1, x_ref, y_ref, dinv_ref):
  deg = d0[...][:, 0:1] + d1[...][:, 0:1] + 1.0
  dinv = lax.rsqrt(deg)
  dinv_ref[...] = dinv
  y_ref[...] = x_ref[...] * dinv


_tc_prep = pl.pallas_call(
    _prep_body,
    grid=(GT,),
    in_specs=[
        pl.BlockSpec((BT, DP), lambda i: (i, 0)),
        pl.BlockSpec((BT, DP), lambda i: (i, 0)),
        pl.BlockSpec((BT, D_IN), lambda i: (i, 0)),
    ],
    out_specs=[
        pl.BlockSpec((BT, D_IN), lambda i: (i, 0)),
        pl.BlockSpec((BT, 1), lambda i: (i, 0)),
    ],
    out_shape=[
        jax.ShapeDtypeStruct((N, D_IN), jnp.float32),
        jax.ShapeDtypeStruct((N, 1), jnp.float32),
    ],
)


def _mid_body(a0, a1, y_ref, dinv_ref, w1, b1r, w2, y2_ref):
  dinv = dinv_ref[...]
  agg = dinv * (a0[...] + a1[...] + y_ref[...])
  x1 = jnp.dot(agg, w1[...], preferred_element_type=jnp.float32) + b1r[...]
  x1 = jnp.where(x1 >= 0.0, x1, 0.01 * x1)
  h2 = jnp.dot(x1, w2[...], preferred_element_type=jnp.float32)
  y2_ref[...] = dinv * h2


_tc_mid = pl.pallas_call(
    _mid_body,
    grid=(GT,),
    in_specs=[
        pl.BlockSpec((BT, D_IN), lambda i: (i, 0)),
        pl.BlockSpec((BT, D_IN), lambda i: (i, 0)),
        pl.BlockSpec((BT, D_IN), lambda i: (i, 0)),
        pl.BlockSpec((BT, 1), lambda i: (i, 0)),
        pl.BlockSpec((D_IN, D_HID), lambda i: (0, 0)),
        pl.BlockSpec((1, D_HID), lambda i: (0, 0)),
        pl.BlockSpec((D_HID, DP), lambda i: (0, 0)),
    ],
    out_specs=pl.BlockSpec((BT, DP), lambda i: (i, 0)),
    out_shape=jax.ShapeDtypeStruct((N, DP), jnp.float32),
)


def _final_body(a0, a1, y2_ref, dinv_ref, b2r, out_ref):
  z = dinv_ref[...] * (a0[...] + a1[...] + y2_ref[...]) + b2r[...]
  z4 = z[:, 0:NCLS]
  m = jnp.max(z4, axis=1, keepdims=True)
  e = jnp.exp(z4 - m)
  lse = jnp.log(jnp.sum(e, axis=1, keepdims=True))
  out_ref[...] = z4 - m - lse


_tc_final = pl.pallas_call(
    _final_body,
    grid=(GT,),
    in_specs=[
        pl.BlockSpec((BT, DP), lambda i: (i, 0)),
        pl.BlockSpec((BT, DP), lambda i: (i, 0)),
        pl.BlockSpec((BT, DP), lambda i: (i, 0)),
        pl.BlockSpec((BT, 1), lambda i: (i, 0)),
        pl.BlockSpec((1, DP), lambda i: (0, 0)),
    ],
    out_specs=pl.BlockSpec((BT, NCLS), lambda i: (i, 0)),
    out_shape=jax.ShapeDtypeStruct((N, NCLS), jnp.float32),
)


def _pad_edges(adj):
  src = jnp.concatenate(
      [adj[0].astype(jnp.int32), jnp.zeros((EP - E,), jnp.int32)])
  dst = jnp.concatenate(
      [adj[1].astype(jnp.int32), jnp.full((EP - E,), N, jnp.int32)])
  return src.reshape(NW, NCH, C), dst.reshape(NW, NCH, C)


def kernel(in_feat, adj, W1, b1, W2, b2):
  src, dst = _pad_edges(adj)
  deg_parts = _sc_deg(dst)
  y, dinv = _tc_prep(deg_parts[0], deg_parts[1], in_feat)
  acc = _sc_agg128(y, src, dst)
  W2p = jnp.pad(W2, ((0, 0), (0, DP - NCLS)))
  y2 = _tc_mid(acc[0], acc[1], y, dinv, W1, b1.reshape(1, D_HID), W2p)
  acc2 = _sc_agg16(y2, src, dst)
  b2p = jnp.pad(b2, (0, DP - NCLS)).reshape(1, DP)
  return _tc_final(acc2[0], acc2[1], y2, dinv, b2p)


# speedup vs baseline: 20.4433x; 1.0861x over previous
import functools

import jax
import jax.numpy as jnp
from jax import lax
from jax.experimental import pallas as pl
from jax.experimental.pallas import tpu as pltpu
from jax.experimental.pallas import tpu_sc as plsc

N = 10000
E = 320000
D_IN = 128
D_HID = 512
NCLS = 4
DP = 16

NC = 2
NS = 16
NW = NC * NS
C = 128
EP = 327680
EW = EP // NW
NCH = EW // C

NP = 10240
RPS = NP // NS
ZR = 128
NZ = RPS // ZR


def _fill_const(ref, nrows, d, val):
  vec = jnp.full((16,), val, jnp.float32)

  def body(r, carry):
    for cc in range(d // 16):
      ref[r, pl.ds(cc * 16, 16)] = vec
    return carry

  lax.fori_loop(0, nrows, body, 0)


def _make_sc_agg(d, gather):
  mesh = plsc.VectorSubcoreMesh(core_axis_name="c", subcore_axis_name="s")
  out_type = jax.ShapeDtypeStruct((NC, NP, d), jnp.float32)
  scratch_types = [
      pltpu.VMEM((NCH, C), jnp.int32),
      pltpu.VMEM((2, C, d), jnp.float32),
      pltpu.VMEM_SHARED((NP, d), jnp.float32),
      pltpu.SemaphoreType.DMA,
      pltpu.SemaphoreType.DMA,
  ]
  if gather:
    scratch_types.insert(0, pltpu.VMEM((2, C), jnp.int32))

  def body(*refs):
    if gather:
      (tab_hbm, src_hbm, dst_hbm, out_hbm,
       sring, dsti, rows, acc, semg, semi) = refs
    else:
      (dst_hbm, out_hbm, dsti, rows, acc, semg, semi) = refs
    c = lax.axis_index("c")
    s = lax.axis_index("s")
    wid = c * NS + s

    pltpu.sync_copy(dst_hbm.at[wid], dsti)

    _fill_const(rows.at[0], ZR, d, 0.0)

    def zcopy(k, carry):
      pltpu.sync_copy(rows.at[0], acc.at[pl.ds(s * RPS + k * ZR, ZR)])
      return carry

    lax.fori_loop(0, NZ, zcopy, 0)
    plsc.subcore_barrier()

    if gather:
      pltpu.sync_copy(src_hbm.at[wid, 0], sring.at[0])
      pltpu.async_copy(tab_hbm.at[sring.at[0]], rows.at[0], semg)
      pltpu.async_copy(src_hbm.at[wid, 1], sring.at[1], semi)

      def step(j, carry):
        b = j % 2
        nb = (j + 1) % 2
        pltpu.make_async_copy(tab_hbm.at[sring.at[b]], rows.at[b], semg).wait()

        @pl.when(j + 1 < NCH)
        def _():
          pltpu.make_async_copy(
              src_hbm.at[wid, j + 1], sring.at[nb], semi).wait()
          pltpu.async_copy(tab_hbm.at[sring.at[nb]], rows.at[nb], semg)

        @pl.when(j + 2 < NCH)
        def _():
          pltpu.async_copy(src_hbm.at[wid, j + 2], sring.at[b], semi)

        pltpu.sync_copy(rows.at[b], acc.at[dsti.at[j]], add=True)
        return carry
    else:
      _fill_const(rows.at[0], C, d, 1.0)

      def step(j, carry):
        pltpu.sync_copy(rows.at[0], acc.at[dsti.at[j]], add=True)
        return carry

    lax.fori_loop(0, NCH, step, 0)
    plsc.subcore_barrier()

    def ocopy(k, carry):
      r0 = s * RPS + k * ZR
      pltpu.sync_copy(acc.at[pl.ds(r0, ZR)], rows.at[0])
      pltpu.sync_copy(rows.at[0], out_hbm.at[c, pl.ds(r0, ZR)])
      return carry

    lax.fori_loop(0, NZ, ocopy, 0)

  params = pltpu.CompilerParams(use_tc_tiling_on_sc=(d == 128))
  return functools.partial(
      pl.kernel, mesh=mesh, out_type=out_type,
      scratch_types=scratch_types, compiler_params=params)(body)


_sc_deg = _make_sc_agg(DP, gather=False)
_sc_agg128 = _make_sc_agg(D_IN, gather=True)
_sc_agg16 = _make_sc_agg(DP, gather=True)

BT = 400
GT = N // BT


def _prep_body(d0, d1, x_ref, y_ref, dinv_ref):
  deg = d0[...][:, 0:1] + d1[...][:, 0:1] + 1.0
  dinv = lax.rsqrt(deg)
  dinv_ref[...] = dinv
  y_ref[...] = x_ref[...] * dinv


_tc_prep = pl.pallas_call(
    _prep_body,
    grid=(GT,),
    in_specs=[
        pl.BlockSpec((BT, DP), lambda i: (i, 0)),
        pl.BlockSpec((BT, DP), lambda i: (i, 0)),
        pl.BlockSpec((BT, D_IN), lambda i: (i, 0)),
    ],
    out_specs=[
        pl.BlockSpec((BT, D_IN), lambda i: (i, 0)),
        pl.BlockSpec((BT, 1), lambda i: (i, 0)),
    ],
    out_shape=[
        jax.ShapeDtypeStruct((N, D_IN), jnp.float32),
        jax.ShapeDtypeStruct((N, 1), jnp.float32),
    ],
)


def _mid_body(a0, a1, y_ref, dinv_ref, w1, b1r, w2, y2_ref):
  dinv = dinv_ref[...]
  agg = dinv * (a0[...] + a1[...] + y_ref[...])
  x1 = jnp.dot(agg, w1[...], preferred_element_type=jnp.float32) + b1r[...]
  x1 = jnp.where(x1 >= 0.0, x1, 0.01 * x1)
  h2 = jnp.dot(x1, w2[...], preferred_element_type=jnp.float32)
  y2_ref[...] = dinv * h2


_tc_mid = pl.pallas_call(
    _mid_body,
    grid=(GT,),
    in_specs=[
        pl.BlockSpec((BT, D_IN), lambda i: (i, 0)),
        pl.BlockSpec((BT, D_IN), lambda i: (i, 0)),
        pl.BlockSpec((BT, D_IN), lambda i: (i, 0)),
        pl.BlockSpec((BT, 1), lambda i: (i, 0)),
        pl.BlockSpec((D_IN, D_HID), lambda i: (0, 0)),
        pl.BlockSpec((1, D_HID), lambda i: (0, 0)),
        pl.BlockSpec((D_HID, DP), lambda i: (0, 0)),
    ],
    out_specs=pl.BlockSpec((BT, DP), lambda i: (i, 0)),
    out_shape=jax.ShapeDtypeStruct((N, DP), jnp.float32),
)


def _final_body(a0, a1, y2_ref, dinv_ref, b2r, out_ref):
  z = dinv_ref[...] * (a0[...] + a1[...] + y2_ref[...]) + b2r[...]
  z4 = z[:, 0:NCLS]
  m = jnp.max(z4, axis=1, keepdims=True)
  e = jnp.exp(z4 - m)
  lse = jnp.log(jnp.sum(e, axis=1, keepdims=True))
  out_ref[...] = z4 - m - lse


_tc_final = pl.pallas_call(
    _final_body,
    grid=(GT,),
    in_specs=[
        pl.BlockSpec((BT, DP), lambda i: (i, 0)),
        pl.BlockSpec((BT, DP), lambda i: (i, 0)),
        pl.BlockSpec((BT, DP), lambda i: (i, 0)),
        pl.BlockSpec((BT, 1), lambda i: (i, 0)),
        pl.BlockSpec((1, DP), lambda i: (0, 0)),
    ],
    out_specs=pl.BlockSpec((BT, NCLS), lambda i: (i, 0)),
    out_shape=jax.ShapeDtypeStruct((N, NCLS), jnp.float32),
)


def _pad_edges(adj):
  src = jnp.concatenate(
      [adj[0].astype(jnp.int32), jnp.zeros((EP - E,), jnp.int32)])
  dst = jnp.concatenate(
      [adj[1].astype(jnp.int32), jnp.full((EP - E,), N, jnp.int32)])
  return src.reshape(NW, NCH, C), dst.reshape(NW, NCH, C)


def kernel(in_feat, adj, W1, b1, W2, b2):
  src, dst = _pad_edges(adj)
  deg_parts = _sc_deg(dst)
  y, dinv = _tc_prep(deg_parts[0], deg_parts[1], in_feat)
  acc = _sc_agg128(y, src, dst)
  W2p = jnp.pad(W2, ((0, 0), (0, DP - NCLS)))
  y2 = _tc_mid(acc[0], acc[1], y, dinv, W1, b1.reshape(1, D_HID), W2p)
  acc2 = _sc_agg16(y2, src, dst)
  b2p = jnp.pad(b2, (0, DP - NCLS)).reshape(1, DP)
  return _tc_final(acc2[0], acc2[1], y2, dinv, b2p)
